# Initial kernel scaffold; baseline (speedup 1.0000x reference)
import functools

import jax
import jax.numpy as jnp
from jax import lax
from jax.experimental import pallas as pl
from jax.experimental.pallas import tpu as pltpu
from jax.experimental.pallas import tpu_sc as plsc

N = 10000
E = 160000
D_IN = 256
D_OUT = 256
H = 128
NC = 2
NS = 16
EDGES_PER_TILE = E // NS
CHUNK = 80
NCHUNK = EDGES_PER_TILE // CHUNK
ROWS_PER_TILE = N // NS
MM_BLK = 2500


def _linear_kernel(x_ref, w_ref, b_ref, o_ref):
    acc = lax.dot_general(
        x_ref[...], w_ref[...],
        dimension_numbers=(((1,), (1,)), ((), ())),
        preferred_element_type=jnp.float32,
    )
    o_ref[...] = acc + b_ref[...]


def _linear(x, W, b2):
    grid = (NC, N // MM_BLK)
    return pl.pallas_call(
        _linear_kernel,
        grid=grid,
        in_specs=[
            pl.BlockSpec((MM_BLK, D_IN), lambda c, r: (r, 0)),
            pl.BlockSpec((H, D_IN), lambda c, r: (c, 0)),
            pl.BlockSpec((1, H), lambda c, r: (c, 0)),
        ],
        out_specs=pl.BlockSpec((MM_BLK, H), lambda c, r: (c * (N // MM_BLK) + r, 0)),
        out_shape=jax.ShapeDtypeStruct((NC * N, H), jnp.float32),
    )(x, W, b2)


def _spmm_body(support_hbm, col_hbm, row_hbm, val_hbm, zeros_hbm, out_hbm,
               idx_v, ridx_v, vals_v, rows_v, acc, sem):
    c = lax.axis_index("c")
    s = lax.axis_index("s")

    pltpu.sync_copy(zeros_hbm, acc.at[pl.ds(s * ROWS_PER_TILE, ROWS_PER_TILE)])
    plsc.subcore_barrier()

    ebase = s * EDGES_PER_TILE
    coff = c * N

    def chunk_body(g, _):
        off = ebase + g * CHUNK
        pltpu.sync_copy(col_hbm.at[pl.ds(off, CHUNK)], idx_v)
        pltpu.sync_copy(row_hbm.at[pl.ds(off, CHUNK)], ridx_v)
        pltpu.sync_copy(val_hbm.at[pl.ds(off, CHUNK)], vals_v)
        for i in range(CHUNK // 16):
            sl = pl.ds(i * 16, 16)
            idx_v[sl] = idx_v[sl] + coff
        pltpu.async_copy(support_hbm.at[idx_v], rows_v, sem).wait()

        def edge_body(e, _):
            v = vals_v[e]
            for j in range(H // 16):
                sl = pl.ds(j * 16, 16)
                rows_v[e, sl] = rows_v[e, sl] * v
            return 0

        lax.fori_loop(0, CHUNK, edge_body, 0)
        pltpu.sync_copy(rows_v, acc.at[ridx_v], add=True)
        return 0

    lax.fori_loop(0, NCHUNK, chunk_body, 0)
    plsc.subcore_barrier()

    pltpu.sync_copy(
        acc.at[pl.ds(s * ROWS_PER_TILE, ROWS_PER_TILE)],
        out_hbm.at[pl.ds(c * N + s * ROWS_PER_TILE, ROWS_PER_TILE)],
    )


@jax.jit
def _spmm(support, col, row, vals, zeros):
    mesh = plsc.VectorSubcoreMesh(core_axis_name="c", subcore_axis_name="s")
    return pl.kernel(
        _spmm_body,
        out_type=jax.ShapeDtypeStruct((NC * N, H), jnp.float32),
        mesh=mesh,
        scratch_types=[
            pltpu.VMEM((CHUNK,), jnp.int32),
            pltpu.VMEM((CHUNK,), jnp.int32),
            pltpu.VMEM((CHUNK,), jnp.float32),
            pltpu.VMEM((CHUNK, H), jnp.float32),
            pltpu.VMEM_SHARED((N, H), jnp.float32),
            pltpu.SemaphoreType.DMA,
        ],
    )(support, col, row, vals, zeros)


@jax.jit
def kernel(input, adj_indices, adj_values, W, b):
    support = _linear(input, W, b.reshape(NC, H))
    zeros = jnp.zeros((ROWS_PER_TILE, H), jnp.float32)
    out2 = _spmm(support, adj_indices[1], adj_indices[0], adj_values, zeros)
    return jnp.concatenate([out2[:N], out2[N:]], axis=1)


# speedup vs baseline: 2.8811x; 2.8811x over previous
import functools

import jax
import jax.numpy as jnp
from jax import lax
from jax.experimental import pallas as pl
from jax.experimental.pallas import tpu as pltpu
from jax.experimental.pallas import tpu_sc as plsc

N = 10000
E = 160000
D_IN = 256
D_OUT = 256
H = 128
NC = 2
NS = 16
EDGES_PER_TILE = E // NS
CHUNK = 80
NCHUNK = EDGES_PER_TILE // CHUNK
NPAD = 10240
ROWS_PER_TILE = NPAD // NS
MM_BLK = 2000


def _linear_kernel(x_ref, w_ref, b_ref, o_ref):
    acc = lax.dot_general(
        x_ref[...], w_ref[...],
        dimension_numbers=(((1,), (1,)), ((), ())),
        preferred_element_type=jnp.float32,
    )
    o_ref[...] = acc + b_ref[0]


def _linear(x, W, b2):
    grid = (NC, N // MM_BLK)
    return pl.pallas_call(
        _linear_kernel,
        grid=grid,
        in_specs=[
            pl.BlockSpec((MM_BLK, D_IN), lambda c, r: (r, 0)),
            pl.BlockSpec((H, D_IN), lambda c, r: (c, 0)),
            pl.BlockSpec((1, 1, H), lambda c, r: (c, 0, 0)),
        ],
        out_specs=pl.BlockSpec((MM_BLK, H), lambda c, r: (c * (N // MM_BLK) + r, 0)),
        out_shape=jax.ShapeDtypeStruct((NC * N, H), jnp.float32),
    )(x, W, b2)


def _spmm_body(support_hbm, col_hbm, row_hbm, val_hbm, zeros_hbm, out_hbm,
               idx_v, ridx_v, vals_v, rows_v, acc, sem):
    c = lax.axis_index("c")
    s = lax.axis_index("s")

    pltpu.sync_copy(zeros_hbm, acc.at[pl.ds(s * ROWS_PER_TILE, ROWS_PER_TILE)])
    plsc.subcore_barrier()

    ebase = s * EDGES_PER_TILE
    coff = c * N

    def chunk_body(g, _):
        off = ebase + g * CHUNK
        pltpu.sync_copy(col_hbm.at[pl.ds(off, CHUNK)], idx_v)
        pltpu.sync_copy(row_hbm.at[pl.ds(off, CHUNK)], ridx_v)
        pltpu.sync_copy(val_hbm.at[pl.ds(off, CHUNK)], vals_v)
        for i in range(CHUNK // 16):
            sl = pl.ds(i * 16, 16)
            idx_v[sl] = idx_v[sl] + coff
        pltpu.async_copy(support_hbm.at[idx_v], rows_v, sem).wait()

        for g16 in range(CHUNK // 16):
            vv = vals_v[pl.ds(g16 * 16, 16)]
            for lane in range(16):
                e = g16 * 16 + lane
                v = vv[lane]
                for j in range(H // 16):
                    sl = pl.ds(j * 16, 16)
                    rows_v[e, sl] = rows_v[e, sl] * v
        pltpu.sync_copy(rows_v, acc.at[ridx_v], add=True)
        return 0

    lax.fori_loop(0, NCHUNK, chunk_body, 0)
    plsc.subcore_barrier()

    pltpu.sync_copy(
        acc.at[pl.ds(s * ROWS_PER_TILE, ROWS_PER_TILE)],
        out_hbm.at[pl.ds(c * NPAD + s * ROWS_PER_TILE, ROWS_PER_TILE)],
    )


@jax.jit
def _spmm(support, col, row, vals, zeros):
    mesh = plsc.VectorSubcoreMesh(core_axis_name="c", subcore_axis_name="s")
    return pl.kernel(
        _spmm_body,
        out_type=jax.ShapeDtypeStruct((NC * NPAD, H), jnp.float32),
        mesh=mesh,
        scratch_types=[
            pltpu.VMEM((CHUNK,), jnp.int32),
            pltpu.VMEM((CHUNK,), jnp.int32),
            pltpu.VMEM((CHUNK,), jnp.float32),
            pltpu.VMEM((CHUNK, H), jnp.float32),
            pltpu.VMEM_SHARED((NPAD, H), jnp.float32),
            pltpu.SemaphoreType.DMA,
        ],
    )(support, col, row, vals, zeros)


@jax.jit
def kernel(input, adj_indices, adj_values, W, b):
    support = _linear(input, W, b.reshape(NC, 1, H))
    zeros = jnp.zeros((ROWS_PER_TILE, H), jnp.float32)
    out2 = _spmm(support, adj_indices[1], adj_indices[0], adj_values, zeros)
    return jnp.concatenate([out2[:N], out2[NPAD:NPAD + N]], axis=1)


# speedup vs baseline: 5.0905x; 1.7669x over previous
import jax
import jax.numpy as jnp
from jax import lax
from jax.experimental import pallas as pl
from jax.experimental.pallas import tpu as pltpu
from jax.experimental.pallas import tpu_sc as plsc

N = 10000
E = 160000
D_IN = 256
D_OUT = 256
H = 128
NC = 2
NS = 16
EDGES_PER_TILE = E // NS
CHUNK = 80
NCHUNK = EDGES_PER_TILE // CHUNK
NSTEP = 128
NBUF = 4
NPAD = 10240
ROWS_PER_TILE = NPAD // NS
MM_BLK = 2000


def _linear_kernel(x_ref, w_ref, b_ref, o_ref):
    acc = lax.dot_general(
        x_ref[...], w_ref[...],
        dimension_numbers=(((1,), (1,)), ((), ())),
        preferred_element_type=jnp.float32,
    )
    o_ref[...] = acc + b_ref[0]


def _linear(x, W, b2):
    grid = (NC, N // MM_BLK)
    return pl.pallas_call(
        _linear_kernel,
        grid=grid,
        in_specs=[
            pl.BlockSpec((MM_BLK, D_IN), lambda c, r: (r, 0)),
            pl.BlockSpec((H, D_IN), lambda c, r: (c, 0)),
            pl.BlockSpec((1, 1, H), lambda c, r: (c, 0, 0)),
        ],
        out_specs=pl.BlockSpec((MM_BLK, H), lambda c, r: (c * (N // MM_BLK) + r, 0)),
        out_shape=jax.ShapeDtypeStruct((NC * N, H), jnp.float32),
    )(x, W, b2)


def _spmm_body(support_hbm, col_hbm, row_hbm, val_hbm, zeros_hbm, out_hbm,
               idx_v, ridx_v, vals_v, rows_v, acc, csem, gsem, ssem):
    c = lax.axis_index("c")
    s = lax.axis_index("s")

    pltpu.sync_copy(zeros_hbm, acc.at[pl.ds(s * ROWS_PER_TILE, ROWS_PER_TILE)])
    plsc.subcore_barrier()

    ebase = s * EDGES_PER_TILE
    coff = c * N

    def chunk_off(cg):
        return ebase + jnp.minimum(cg, NCHUNK - 1) * CHUNK

    def stage0_start(cg, q):
        off = chunk_off(cg)
        pltpu.async_copy(col_hbm.at[pl.ds(off, CHUNK)], idx_v.at[q], csem.at[q])
        pltpu.async_copy(row_hbm.at[pl.ds(off, CHUNK)], ridx_v.at[q], csem.at[q])
        pltpu.async_copy(val_hbm.at[pl.ds(off, CHUNK)], vals_v.at[q], csem.at[q])

    def stage0_wait(q):
        pltpu.make_async_copy(col_hbm.at[pl.ds(0, CHUNK)], idx_v.at[q], csem.at[q]).wait()
        pltpu.make_async_copy(row_hbm.at[pl.ds(0, CHUNK)], ridx_v.at[q], csem.at[q]).wait()
        pltpu.make_async_copy(val_hbm.at[pl.ds(0, CHUNK)], vals_v.at[q], csem.at[q]).wait()

    def gather_start(q):
        for i in range(CHUNK // 16):
            sl = pl.ds(i * 16, 16)
            idx_v[q, sl] = idx_v[q, sl] + coff
        pltpu.async_copy(support_hbm.at[idx_v.at[q]], rows_v.at[q], gsem.at[q])

    def gather_wait(q):
        pltpu.make_async_copy(support_hbm.at[idx_v.at[q]], rows_v.at[q], gsem.at[q]).wait()

    def scatter_start(q):
        pltpu.async_copy(rows_v.at[q], acc.at[ridx_v.at[q]], ssem.at[q], add=True)

    def scatter_wait(q):
        pltpu.make_async_copy(rows_v.at[q], acc.at[ridx_v.at[q]], ssem.at[q]).wait()

    def multiply(cg, q):
        m = jnp.where(cg < NCHUNK, 1.0, 0.0).astype(jnp.float32)

        def g16_body(i, _):
            vv = vals_v[q, pl.ds(i * 16, 16)] * m
            for lane in range(16):
                e = i * 16 + lane
                v = vv[lane]
                for j in range(H // 16):
                    sl = pl.ds(j * 16, 16)
                    rows_v[q, e, sl] = rows_v[q, e, sl] * v
            return 0

        lax.fori_loop(0, CHUNK // 16, g16_body, 0)

    for q in range(3):
        stage0_start(q, q)
    stage0_wait(0)
    gather_start(0)

    def step(cg, k, it):
        q, q1, q3 = k & 3, (k + 1) & 3, (k + 3) & 3
        gather_wait(q)
        multiply(cg, q)
        scatter_start(q)
        if k == 0:
            @pl.when(it >= 1)
            def _():
                scatter_wait(q3)
        else:
            scatter_wait(q3)

        def refill():
            stage0_start(cg + 3, q3)
        if k == 0:
            refill()
        else:
            pl.when(it < NSTEP // 4 - 1)(refill)

        def next_gather():
            stage0_wait(q1)
            gather_start(q1)
        if k < 3:
            next_gather()
        else:
            pl.when(it < NSTEP // 4 - 1)(next_gather)

    def loop_body(it, _):
        for k in range(4):
            step(it * 4 + k, k, it)
        return 0

    lax.fori_loop(0, NSTEP // 4, loop_body, 0)
    scatter_wait(3)

    plsc.subcore_barrier()
    pltpu.sync_copy(
        acc.at[pl.ds(s * ROWS_PER_TILE, ROWS_PER_TILE)],
        out_hbm.at[pl.ds(c * NPAD + s * ROWS_PER_TILE, ROWS_PER_TILE)],
    )


@jax.jit
def _spmm(support, col, row, vals, zeros):
    mesh = plsc.VectorSubcoreMesh(core_axis_name="c", subcore_axis_name="s")
    return pl.kernel(
        _spmm_body,
        out_type=jax.ShapeDtypeStruct((NC * NPAD, H), jnp.float32),
        mesh=mesh,
        scratch_types=[
            pltpu.VMEM((NBUF, CHUNK), jnp.int32),
            pltpu.VMEM((NBUF, CHUNK), jnp.int32),
            pltpu.VMEM((NBUF, CHUNK), jnp.float32),
            pltpu.VMEM((NBUF, CHUNK, H), jnp.float32),
            pltpu.VMEM_SHARED((NPAD, H), jnp.float32),
            pltpu.SemaphoreType.DMA((NBUF,)),
            pltpu.SemaphoreType.DMA((NBUF,)),
            pltpu.SemaphoreType.DMA((NBUF,)),
        ],
    )(support, col, row, vals, zeros)


@jax.jit
def kernel(input, adj_indices, adj_values, W, b):
    support = _linear(input, W, b.reshape(NC, 1, H))
    zeros = jnp.zeros((ROWS_PER_TILE, H), jnp.float32)
    out2 = _spmm(support, adj_indices[1], adj_indices[0], adj_values, zeros)
    return jnp.concatenate([out2[:N], out2[NPAD:NPAD + N]], axis=1)


# speedup vs baseline: 6.9378x; 1.3629x over previous
import jax
import jax.numpy as jnp
from jax import lax
from jax.experimental import pallas as pl
from jax.experimental.pallas import tpu as pltpu
from jax.experimental.pallas import tpu_sc as plsc

N = 10000
E = 160000
D_IN = 256
D_OUT = 256
H = 128
NC = 2
NS = 16
EDGES_PER_TILE = E // NS
CHUNK = 80
NCHUNK = EDGES_PER_TILE // CHUNK
NSTEP = 128
NBUF = 4
NPAD = 10240
ROWS_PER_TILE = NPAD // NS
MM_BLK = 2000


def _linear_kernel(x_ref, w_ref, b_ref, o_ref):
    acc = lax.dot_general(
        x_ref[...], w_ref[...],
        dimension_numbers=(((1,), (1,)), ((), ())),
        preferred_element_type=jnp.float32,
    )
    o_ref[...] = acc + b_ref[0]


def _linear(x, W, b2):
    grid = (NC, N // MM_BLK)
    return pl.pallas_call(
        _linear_kernel,
        grid=grid,
        in_specs=[
            pl.BlockSpec((MM_BLK, D_IN), lambda c, r: (r, 0)),
            pl.BlockSpec((H, D_IN), lambda c, r: (c, 0)),
            pl.BlockSpec((1, 1, H), lambda c, r: (c, 0, 0)),
        ],
        out_specs=pl.BlockSpec((MM_BLK, H), lambda c, r: (c * (N // MM_BLK) + r, 0)),
        out_shape=jax.ShapeDtypeStruct((NC * N, H), jnp.float32),
    )(x, W, b2)


def _spmm_body(support_hbm, col_hbm, row_hbm, val_hbm, zeros_hbm, out_hbm,
               idx_v, ridx_v, vals_v, rows_v, acc, csem, gsem, ssem):
    c = lax.axis_index("c")
    s = lax.axis_index("s")

    pltpu.sync_copy(zeros_hbm, acc.at[pl.ds(s * ROWS_PER_TILE, ROWS_PER_TILE)])
    plsc.subcore_barrier()

    ebase = s * EDGES_PER_TILE
    coff = c * N

    def chunk_off(cg):
        return ebase + jnp.minimum(cg, NCHUNK - 1) * CHUNK

    def stage0_start(cg, q):
        off = chunk_off(cg)
        pltpu.async_copy(col_hbm.at[pl.ds(off, CHUNK)], idx_v.at[q], csem.at[q])
        pltpu.async_copy(row_hbm.at[pl.ds(off, CHUNK)], ridx_v.at[q], csem.at[q])
        pltpu.async_copy(val_hbm.at[pl.ds(off, CHUNK)], vals_v.at[q], csem.at[q])

    def stage0_wait(q):
        pltpu.make_async_copy(col_hbm.at[pl.ds(0, CHUNK)], idx_v.at[q], csem.at[q]).wait()
        pltpu.make_async_copy(row_hbm.at[pl.ds(0, CHUNK)], ridx_v.at[q], csem.at[q]).wait()
        pltpu.make_async_copy(val_hbm.at[pl.ds(0, CHUNK)], vals_v.at[q], csem.at[q]).wait()

    def gather_start(q):
        for i in range(CHUNK // 16):
            sl = pl.ds(i * 16, 16)
            idx_v[q, sl] = idx_v[q, sl] + coff
        pltpu.async_copy(support_hbm.at[idx_v.at[q]], rows_v.at[q], gsem.at[q])

    def gather_wait(q):
        pltpu.make_async_copy(support_hbm.at[idx_v.at[q]], rows_v.at[q], gsem.at[q]).wait()

    def scatter_start(q):
        pltpu.async_copy(rows_v.at[q], acc.at[ridx_v.at[q]], ssem.at[q], add=True)

    def scatter_wait(q):
        pltpu.make_async_copy(rows_v.at[q], acc.at[ridx_v.at[q]], ssem.at[q]).wait()

    def multiply(cg, q):
        m = jnp.where(cg < NCHUNK, 1.0, 0.0).astype(jnp.float32)

        def g16_body(i, _):
            vv = vals_v[q, pl.ds(i * 16, 16)] * m
            for lane in range(16):
                e = i * 16 + lane
                v = vv[lane]
                for j in range(H // 16):
                    sl = pl.ds(j * 16, 16)
                    rows_v[q, e, sl] = rows_v[q, e, sl] * v
            return 0

        lax.fori_loop(0, CHUNK // 16, g16_body, 0)

    for q in range(3):
        stage0_start(q, q)
    stage0_wait(0)
    gather_start(0)
    stage0_wait(1)
    gather_start(1)

    last_it = NSTEP // 4 - 1

    def step(cg, k, it):
        q, q1, q2, q3 = k & 3, (k + 1) & 3, (k + 2) & 3, (k + 3) & 3
        gather_wait(q)

        def next_gather():
            stage0_wait(q2)
            gather_start(q2)
        if k < 2:
            next_gather()
        else:
            pl.when(it < last_it)(next_gather)

        multiply(cg, q)
        scatter_start(q)

        if k == 0:
            @pl.when(it >= 1)
            def _():
                scatter_wait(q3)
        else:
            scatter_wait(q3)

        def refill():
            stage0_start(cg + 3, q3)
        if k == 0:
            refill()
        else:
            pl.when(it < last_it)(refill)

    def loop_body(it, _):
        for k in range(4):
            step(it * 4 + k, k, it)
        return 0

    lax.fori_loop(0, NSTEP // 4, loop_body, 0)
    scatter_wait(3)

    plsc.subcore_barrier()
    pltpu.sync_copy(
        acc.at[pl.ds(s * ROWS_PER_TILE, ROWS_PER_TILE)],
        out_hbm.at[pl.ds(s * ROWS_PER_TILE, ROWS_PER_TILE),
                   pl.ds(pl.multiple_of(c * H, H), H)],
    )


@jax.jit
def _spmm(support, col, row, vals, zeros):
    mesh = plsc.VectorSubcoreMesh(core_axis_name="c", subcore_axis_name="s")
    return pl.kernel(
        _spmm_body,
        out_type=jax.ShapeDtypeStruct((NPAD, D_OUT), jnp.float32),
        mesh=mesh,
        scratch_types=[
            pltpu.VMEM((NBUF, CHUNK), jnp.int32),
            pltpu.VMEM((NBUF, CHUNK), jnp.int32),
            pltpu.VMEM((NBUF, CHUNK), jnp.float32),
            pltpu.VMEM((NBUF, CHUNK, H), jnp.float32),
            pltpu.VMEM_SHARED((NPAD, H), jnp.float32),
            pltpu.SemaphoreType.DMA((NBUF,)),
            pltpu.SemaphoreType.DMA((NBUF,)),
            pltpu.SemaphoreType.DMA((NBUF,)),
        ],
    )(support, col, row, vals, zeros)


@jax.jit
def kernel(input, adj_indices, adj_values, W, b):
    support = _linear(input, W, b.reshape(NC, 1, H))
    zeros = jnp.zeros((ROWS_PER_TILE, H), jnp.float32)
    out2 = _spmm(support, adj_indices[1], adj_indices[0], adj_values, zeros)
    return out2[:N]
